# baseline (device time: 125019 ns/iter reference)
import jax
import jax.numpy as jnp
from jax import lax
from jax.experimental import pallas as pl
from jax.experimental.pallas import tpu as pltpu

N_DEV = 4
M = 4096
D = 1024
F = 4096
M_BLK = M // N_DEV
D_HALF = D // 2

BF = jnp.bfloat16
F32 = jnp.float32


def _rows(c):
    return pl.ds(c * M_BLK, M_BLK)


def _body(x_ref, w1_ref, w2_ref, out_ref, h_buf, rs_r, rs_l,
          sems_sr, sems_rr, sems_sl, sems_rl,
          ag_sr, ag_rr, ag_sl, ag_rl):
    my = lax.axis_index("i")
    left = (my - 1) % N_DEV
    right = (my + 1) % N_DEV
    CR = pl.ds(0, D_HALF)
    CL = pl.ds(D_HALF, D_HALF)

    barrier_sem = pltpu.get_barrier_semaphore()
    for nbr in (left, right):
        pl.semaphore_signal(
            barrier_sem, inc=1,
            device_id=(nbr,), device_id_type=pl.DeviceIdType.MESH,
        )
    pl.semaphore_wait(barrier_sem, 2)

    def gemm1(c):
        h_buf[_rows(c), :] = jnp.dot(
            x_ref[_rows(c), :], w1_ref[:, :], preferred_element_type=F32
        ).astype(BF)

    def rs_send(s, src_r, src_l):
        r = pltpu.make_async_remote_copy(
            src_ref=src_r, dst_ref=rs_r.at[s],
            send_sem=sems_sr.at[s], recv_sem=sems_rr.at[s],
            device_id=(right,), device_id_type=pl.DeviceIdType.MESH,
        )
        l = pltpu.make_async_remote_copy(
            src_ref=src_l, dst_ref=rs_l.at[s],
            send_sem=sems_sl.at[s], recv_sem=sems_rl.at[s],
            device_id=(left,), device_id_type=pl.DeviceIdType.MESH,
        )
        r.start()
        l.start()
        return r, l

    gemm1(my % N_DEV)
    s0_r, s0_l = rs_send(0, h_buf.at[_rows(my), CR], h_buf.at[_rows(my), CL])
    gemm1((my + 1) % N_DEV)
    gemm1((my - 1) % N_DEV)
    gemm1((my + 2) % N_DEV)

    s0_r.wait()
    rs_r[0, :, :] = rs_r[0, :, :] + h_buf[_rows((my - 1) % N_DEV), CR]
    s1_r, _sl1 = None, None
    s0_l.wait()
    rs_l[0, :, :] = rs_l[0, :, :] + h_buf[_rows((my + 1) % N_DEV), CL]
    s1_r, s1_l = rs_send(1, rs_r.at[0], rs_l.at[0])

    s1_r.wait()
    rs_r[1, :, :] = rs_r[1, :, :] + h_buf[_rows((my - 2) % N_DEV), CR]
    s1_l.wait()
    rs_l[1, :, :] = rs_l[1, :, :] + h_buf[_rows((my + 2) % N_DEV), CL]
    s2_r, s2_l = rs_send(2, rs_r.at[1], rs_l.at[1])

    s2_r.wait()
    rs_r[2, :, :] = rs_r[2, :, :] + h_buf[_rows((my + 1) % N_DEV), CR]
    s2_l.wait()
    rs_l[2, :, :] = rs_l[2, :, :] + h_buf[_rows((my - 1) % N_DEV), CL]

    w2_top = w2_ref[0:D_HALF, :]
    w2_bot = w2_ref[D_HALF:D, :]

    def ag_send(t, ch_r, ch_l, src_r, src_l):
        r = pltpu.make_async_remote_copy(
            src_ref=src_r, dst_ref=h_buf.at[_rows(ch_r), CR],
            send_sem=ag_sr.at[t], recv_sem=ag_rr.at[t],
            device_id=(right,), device_id_type=pl.DeviceIdType.MESH,
        )
        l = pltpu.make_async_remote_copy(
            src_ref=src_l, dst_ref=h_buf.at[_rows(ch_l), CL],
            send_sem=ag_sl.at[t], recv_sem=ag_rl.at[t],
            device_id=(left,), device_id_type=pl.DeviceIdType.MESH,
        )
        r.start()
        l.start()
        return r, l

    t0_r, t0_l = ag_send(
        0, (my + 1) % N_DEV, (my - 1) % N_DEV, rs_r.at[2], rs_l.at[2]
    )
    out_ref[_rows((my + 1) % N_DEV), :] = jnp.dot(
        rs_r[2, :, :], w2_top, preferred_element_type=F32
    )
    out_ref[_rows((my - 1) % N_DEV), :] = jnp.dot(
        rs_l[2, :, :], w2_bot, preferred_element_type=F32
    )

    t0_r.wait()
    t0_l.wait()
    t1_r, t1_l = ag_send(
        1, my % N_DEV, my % N_DEV,
        h_buf.at[_rows(my), CR], h_buf.at[_rows(my), CL],
    )
    out_ref[_rows(my), :] = jnp.dot(
        h_buf[_rows(my), CR], w2_top, preferred_element_type=F32
    )
    out_ref[_rows(my), :] = out_ref[_rows(my), :] + jnp.dot(
        h_buf[_rows(my), CL], w2_bot, preferred_element_type=F32
    )

    t1_r.wait()
    t1_l.wait()
    t2_r, t2_l = ag_send(
        2, (my - 1) % N_DEV, (my + 1) % N_DEV,
        h_buf.at[_rows((my - 1) % N_DEV), CR],
        h_buf.at[_rows((my + 1) % N_DEV), CL],
    )
    out_ref[_rows((my - 1) % N_DEV), :] = (
        out_ref[_rows((my - 1) % N_DEV), :]
        + jnp.dot(h_buf[_rows((my - 1) % N_DEV), CR], w2_top,
                  preferred_element_type=F32)
    )
    out_ref[_rows((my + 1) % N_DEV), :] = (
        out_ref[_rows((my + 1) % N_DEV), :]
        + jnp.dot(h_buf[_rows((my + 1) % N_DEV), CL], w2_bot,
                  preferred_element_type=F32)
    )

    t2_r.wait()
    t2_l.wait()
    c2 = (my + 2) % N_DEV
    out_ref[_rows(c2), :] = jnp.dot(
        h_buf[_rows(c2), CR], w2_top, preferred_element_type=F32
    )
    out_ref[_rows(c2), :] = out_ref[_rows(c2), :] + jnp.dot(
        h_buf[_rows(c2), CL], w2_bot, preferred_element_type=F32
    )


def kernel(x, W1, W2):
    xb = x.astype(BF)
    W1b = W1.astype(BF)
    W2b = W2.astype(BF)

    sem3 = pltpu.SemaphoreType.DMA((N_DEV - 1,))
    return pl.pallas_call(
        _body,
        out_shape=jax.ShapeDtypeStruct((M, D), F32),
        in_specs=[
            pl.BlockSpec(memory_space=pltpu.VMEM),
            pl.BlockSpec(memory_space=pltpu.VMEM),
            pl.BlockSpec(memory_space=pltpu.VMEM),
        ],
        out_specs=pl.BlockSpec(memory_space=pltpu.VMEM),
        scratch_shapes=[
            pltpu.VMEM((M, D), BF),
            pltpu.VMEM((N_DEV - 1, M_BLK, D_HALF), BF),
            pltpu.VMEM((N_DEV - 1, M_BLK, D_HALF), BF),
            sem3, sem3, sem3, sem3,
            sem3, sem3, sem3, sem3,
        ],
        compiler_params=pltpu.CompilerParams(
            collective_id=0, vmem_limit_bytes=100 * 1024 * 1024
        ),
    )(xb, W1b, W2b)


# device time: 121011 ns/iter; 1.0331x vs baseline; 1.0331x over previous
import jax
import jax.numpy as jnp
from jax import lax
from jax.experimental import pallas as pl
from jax.experimental.pallas import tpu as pltpu

N_DEV = 4
M = 4096
D = 1024
M_BLK = M // N_DEV
D_HALF = D // 2


def _rows(c):
    return pl.ds(c * M_BLK, M_BLK)


def _allreduce_body(h_ref, out_ref, rs_r, rs_l,
                    sems_sr, sems_rr, sems_sl, sems_rl,
                    ag_sr, ag_rr, ag_sl, ag_rl):
    my = lax.axis_index("i")
    left = (my - 1) % N_DEV
    right = (my + 1) % N_DEV
    CR = pl.ds(0, D_HALF)
    CL = pl.ds(D_HALF, D_HALF)

    barrier_sem = pltpu.get_barrier_semaphore()
    for nbr in (left, right):
        pl.semaphore_signal(
            barrier_sem, inc=1,
            device_id=(nbr,), device_id_type=pl.DeviceIdType.MESH,
        )
    pl.semaphore_wait(barrier_sem, 2)

    def rs_send(s, src_r, src_l):
        r = pltpu.make_async_remote_copy(
            src_ref=src_r, dst_ref=rs_r.at[s],
            send_sem=sems_sr.at[s], recv_sem=sems_rr.at[s],
            device_id=(right,), device_id_type=pl.DeviceIdType.MESH,
        )
        l = pltpu.make_async_remote_copy(
            src_ref=src_l, dst_ref=rs_l.at[s],
            send_sem=sems_sl.at[s], recv_sem=sems_rl.at[s],
            device_id=(left,), device_id_type=pl.DeviceIdType.MESH,
        )
        r.start()
        l.start()
        return r, l

    s0_r, s0_l = rs_send(0, h_ref.at[_rows(my), CR], h_ref.at[_rows(my), CL])

    s0_r.wait()
    rs_r[0, :, :] = rs_r[0, :, :] + h_ref[_rows((my - 1) % N_DEV), CR]
    s0_l.wait()
    rs_l[0, :, :] = rs_l[0, :, :] + h_ref[_rows((my + 1) % N_DEV), CL]
    s1_r, s1_l = rs_send(1, rs_r.at[0], rs_l.at[0])

    s1_r.wait()
    rs_r[1, :, :] = rs_r[1, :, :] + h_ref[_rows((my - 2) % N_DEV), CR]
    s1_l.wait()
    rs_l[1, :, :] = rs_l[1, :, :] + h_ref[_rows((my + 2) % N_DEV), CL]
    s2_r, s2_l = rs_send(2, rs_r.at[1], rs_l.at[1])

    s2_r.wait()
    rs_r[2, :, :] = rs_r[2, :, :] + h_ref[_rows((my + 1) % N_DEV), CR]
    s2_l.wait()
    rs_l[2, :, :] = rs_l[2, :, :] + h_ref[_rows((my - 1) % N_DEV), CL]

    def ag_send(t, ch_r, ch_l, src_r, src_l):
        r = pltpu.make_async_remote_copy(
            src_ref=src_r, dst_ref=out_ref.at[_rows(ch_r), CR],
            send_sem=ag_sr.at[t], recv_sem=ag_rr.at[t],
            device_id=(right,), device_id_type=pl.DeviceIdType.MESH,
        )
        l = pltpu.make_async_remote_copy(
            src_ref=src_l, dst_ref=out_ref.at[_rows(ch_l), CL],
            send_sem=ag_sl.at[t], recv_sem=ag_rl.at[t],
            device_id=(left,), device_id_type=pl.DeviceIdType.MESH,
        )
        r.start()
        l.start()
        return r, l

    t0_r, t0_l = ag_send(
        0, (my + 1) % N_DEV, (my - 1) % N_DEV, rs_r.at[2], rs_l.at[2]
    )
    out_ref[_rows((my + 1) % N_DEV), CR] = rs_r[2, :, :]
    out_ref[_rows((my - 1) % N_DEV), CL] = rs_l[2, :, :]

    t0_r.wait()
    t0_l.wait()
    t1_r, t1_l = ag_send(
        1, my % N_DEV, my % N_DEV,
        out_ref.at[_rows(my), CR], out_ref.at[_rows(my), CL],
    )

    t1_r.wait()
    t1_l.wait()
    t2_r, t2_l = ag_send(
        2, (my - 1) % N_DEV, (my + 1) % N_DEV,
        out_ref.at[_rows((my - 1) % N_DEV), CR],
        out_ref.at[_rows((my + 1) % N_DEV), CL],
    )

    t2_r.wait()
    t2_l.wait()


def _allreduce(h_partial):
    sem3 = pltpu.SemaphoreType.DMA((N_DEV - 1,))
    return pl.pallas_call(
        _allreduce_body,
        out_shape=jax.ShapeDtypeStruct((M, D), h_partial.dtype),
        in_specs=[pl.BlockSpec(memory_space=pltpu.VMEM)],
        out_specs=pl.BlockSpec(memory_space=pltpu.VMEM),
        scratch_shapes=[
            pltpu.VMEM((N_DEV - 1, M_BLK, D_HALF), h_partial.dtype),
            pltpu.VMEM((N_DEV - 1, M_BLK, D_HALF), h_partial.dtype),
            sem3, sem3, sem3, sem3,
            sem3, sem3, sem3, sem3,
        ],
        compiler_params=pltpu.CompilerParams(
            collective_id=0, vmem_limit_bytes=100 * 1024 * 1024
        ),
    )(h_partial)


def kernel(x, W1, W2):
    xb = x.astype(jnp.bfloat16)
    W1b = W1.astype(jnp.bfloat16)
    W2b = W2.astype(jnp.bfloat16)

    h_partial = jnp.dot(
        xb, W1b, preferred_element_type=jnp.float32
    ).astype(jnp.bfloat16)

    h = _allreduce(h_partial)

    out = jnp.dot(h, W2b, preferred_element_type=jnp.float32)
    return out


# device time: 107973 ns/iter; 1.1579x vs baseline; 1.1208x over previous
import jax
import jax.numpy as jnp
from jax import lax
from jax.experimental import pallas as pl
from jax.experimental.pallas import tpu as pltpu

N_DEV = 4
M = 4096
D = 1024
M_BLK = M // N_DEV
D_HALF = D // 2


def _allreduce_body(h_ref, out_ref, rs_r, rs_l,
                    sems_sr, sems_rr, sems_sl, sems_rl,
                    ag_sr, ag_rr, ag_sl, ag_rl):
    my = lax.axis_index("i")
    left = (my - 1) % N_DEV
    right = (my + 1) % N_DEV

    barrier_sem = pltpu.get_barrier_semaphore()
    for nbr in (left, right):
        pl.semaphore_signal(
            barrier_sem, inc=1,
            device_id=(nbr,), device_id_type=pl.DeviceIdType.MESH,
        )
    pl.semaphore_wait(barrier_sem, 2)

    for s in range(N_DEV - 1):
        ch_r = (my - s) % N_DEV
        ch_l = (my + s) % N_DEV
        if s == 0:
            src_r = h_ref.at[pl.ds(ch_r * M_BLK, M_BLK), pl.ds(0, D_HALF)]
            src_l = h_ref.at[pl.ds(ch_l * M_BLK, M_BLK), pl.ds(D_HALF, D_HALF)]
        else:
            src_r = rs_r.at[s - 1]
            src_l = rs_l.at[s - 1]
        rdma_r = pltpu.make_async_remote_copy(
            src_ref=src_r, dst_ref=rs_r.at[s],
            send_sem=sems_sr.at[s], recv_sem=sems_rr.at[s],
            device_id=(right,), device_id_type=pl.DeviceIdType.MESH,
        )
        rdma_l = pltpu.make_async_remote_copy(
            src_ref=src_l, dst_ref=rs_l.at[s],
            send_sem=sems_sl.at[s], recv_sem=sems_rl.at[s],
            device_id=(left,), device_id_type=pl.DeviceIdType.MESH,
        )
        rdma_r.start()
        rdma_l.start()
        rdma_r.wait()
        rdma_l.wait()

        rcv_r = (my - s - 1) % N_DEV
        rcv_l = (my + s + 1) % N_DEV
        rs_r[s, :, :] = (
            rs_r[s, :, :] + h_ref[pl.ds(rcv_r * M_BLK, M_BLK), pl.ds(0, D_HALF)]
        )
        rs_l[s, :, :] = (
            rs_l[s, :, :]
            + h_ref[pl.ds(rcv_l * M_BLK, M_BLK), pl.ds(D_HALF, D_HALF)]
        )

    own_r = (my + 1) % N_DEV
    own_l = (my - 1) % N_DEV
    out_ref[pl.ds(own_r * M_BLK, M_BLK), pl.ds(0, D_HALF)] = rs_r[N_DEV - 2]
    out_ref[pl.ds(own_l * M_BLK, M_BLK), pl.ds(D_HALF, D_HALF)] = rs_l[N_DEV - 2]

    for t in range(N_DEV - 1):
        ch_r = (my + 1 - t) % N_DEV
        ch_l = (my - 1 + t) % N_DEV
        if t == 0:
            src_r = rs_r.at[N_DEV - 2]
            src_l = rs_l.at[N_DEV - 2]
        else:
            src_r = out_ref.at[pl.ds(ch_r * M_BLK, M_BLK), pl.ds(0, D_HALF)]
            src_l = out_ref.at[pl.ds(ch_l * M_BLK, M_BLK), pl.ds(D_HALF, D_HALF)]
        rdma_r = pltpu.make_async_remote_copy(
            src_ref=src_r,
            dst_ref=out_ref.at[pl.ds(ch_r * M_BLK, M_BLK), pl.ds(0, D_HALF)],
            send_sem=ag_sr.at[t], recv_sem=ag_rr.at[t],
            device_id=(right,), device_id_type=pl.DeviceIdType.MESH,
        )
        rdma_l = pltpu.make_async_remote_copy(
            src_ref=src_l,
            dst_ref=out_ref.at[pl.ds(ch_l * M_BLK, M_BLK), pl.ds(D_HALF, D_HALF)],
            send_sem=ag_sl.at[t], recv_sem=ag_rl.at[t],
            device_id=(left,), device_id_type=pl.DeviceIdType.MESH,
        )
        rdma_r.start()
        rdma_l.start()
        rdma_r.wait()
        rdma_l.wait()


def _allreduce(h_partial):
    sem3 = pltpu.SemaphoreType.DMA((N_DEV - 1,))
    return pl.pallas_call(
        _allreduce_body,
        out_shape=jax.ShapeDtypeStruct((M, D), h_partial.dtype),
        in_specs=[pl.BlockSpec(memory_space=pltpu.VMEM)],
        out_specs=pl.BlockSpec(memory_space=pltpu.VMEM),
        scratch_shapes=[
            pltpu.VMEM((N_DEV - 1, M_BLK, D_HALF), h_partial.dtype),
            pltpu.VMEM((N_DEV - 1, M_BLK, D_HALF), h_partial.dtype),
            sem3, sem3, sem3, sem3,
            sem3, sem3, sem3, sem3,
        ],
        compiler_params=pltpu.CompilerParams(collective_id=0),
    )(h_partial)


def kernel(x, W1, W2):
    xb = x.astype(jnp.bfloat16)
    W1b = W1.astype(jnp.bfloat16)
    W2b = W2.astype(jnp.bfloat16)

    h_partial = jnp.dot(
        xb, W1b, preferred_element_type=jnp.float32
    ).astype(jnp.bfloat16)

    h = _allreduce(h_partial)

    out = jnp.dot(h, W2b, preferred_element_type=jnp.float32)
    return out


# device time: 100097 ns/iter; 1.2490x vs baseline; 1.0787x over previous
import jax
import jax.numpy as jnp
from jax import lax
from jax.experimental import pallas as pl
from jax.experimental.pallas import tpu as pltpu

N_DEV = 4
M = 4096
D = 1024
M_BLK = M // N_DEV
D_HALF = D // 2

BF = jnp.bfloat16
F32 = jnp.float32


def _rows(c):
    return pl.ds(c * M_BLK, M_BLK)


def _body(x_ref, w1_ref, w2_ref, out_ref, h_buf, rs_r, rs_l,
          sems_sr, sems_rr, sems_sl, sems_rl,
          ag_sr, ag_rr, ag_sl, ag_rl):
    my = lax.axis_index("i")
    left = (my - 1) % N_DEV
    right = (my + 1) % N_DEV
    CR = pl.ds(0, D_HALF)
    CL = pl.ds(D_HALF, D_HALF)

    barrier_sem = pltpu.get_barrier_semaphore()
    for nbr in (left, right):
        pl.semaphore_signal(
            barrier_sem, inc=1,
            device_id=(nbr,), device_id_type=pl.DeviceIdType.MESH,
        )
    pl.semaphore_wait(barrier_sem, 2)

    def gemm1(c):
        h_buf[_rows(c), :] = jnp.dot(
            x_ref[_rows(c), :], w1_ref[:, :], preferred_element_type=F32
        ).astype(BF)

    def rs_send(s, src_r, src_l):
        r = pltpu.make_async_remote_copy(
            src_ref=src_r, dst_ref=rs_r.at[s],
            send_sem=sems_sr.at[s], recv_sem=sems_rr.at[s],
            device_id=(right,), device_id_type=pl.DeviceIdType.MESH,
        )
        l = pltpu.make_async_remote_copy(
            src_ref=src_l, dst_ref=rs_l.at[s],
            send_sem=sems_sl.at[s], recv_sem=sems_rl.at[s],
            device_id=(left,), device_id_type=pl.DeviceIdType.MESH,
        )
        r.start()
        l.start()
        return r, l

    gemm1(my % N_DEV)
    s0_r, s0_l = rs_send(0, h_buf.at[_rows(my), CR], h_buf.at[_rows(my), CL])
    gemm1((my + 1) % N_DEV)
    gemm1((my - 1) % N_DEV)
    gemm1((my + 2) % N_DEV)

    s0_r.wait()
    rs_r[0, :, :] = rs_r[0, :, :] + h_buf[_rows((my - 1) % N_DEV), CR]
    s0_l.wait()
    rs_l[0, :, :] = rs_l[0, :, :] + h_buf[_rows((my + 1) % N_DEV), CL]
    s1_r, s1_l = rs_send(1, rs_r.at[0], rs_l.at[0])

    s1_r.wait()
    rs_r[1, :, :] = rs_r[1, :, :] + h_buf[_rows((my - 2) % N_DEV), CR]
    s1_l.wait()
    rs_l[1, :, :] = rs_l[1, :, :] + h_buf[_rows((my + 2) % N_DEV), CL]
    s2_r, s2_l = rs_send(2, rs_r.at[1], rs_l.at[1])

    s2_r.wait()
    rs_r[2, :, :] = rs_r[2, :, :] + h_buf[_rows((my + 1) % N_DEV), CR]
    s2_l.wait()
    rs_l[2, :, :] = rs_l[2, :, :] + h_buf[_rows((my - 1) % N_DEV), CL]

    w2_top = w2_ref[0:D_HALF, :]
    w2_bot = w2_ref[D_HALF:D, :]

    def ag_send(t, ch_r, ch_l, src_r, src_l):
        r = pltpu.make_async_remote_copy(
            src_ref=src_r, dst_ref=h_buf.at[_rows(ch_r), CR],
            send_sem=ag_sr.at[t], recv_sem=ag_rr.at[t],
            device_id=(right,), device_id_type=pl.DeviceIdType.MESH,
        )
        l = pltpu.make_async_remote_copy(
            src_ref=src_l, dst_ref=h_buf.at[_rows(ch_l), CL],
            send_sem=ag_sl.at[t], recv_sem=ag_rl.at[t],
            device_id=(left,), device_id_type=pl.DeviceIdType.MESH,
        )
        r.start()
        l.start()
        return r, l

    t0_r, t0_l = ag_send(
        0, (my + 1) % N_DEV, (my - 1) % N_DEV, rs_r.at[2], rs_l.at[2]
    )
    out_ref[_rows((my + 1) % N_DEV), :] = jnp.dot(
        rs_r[2, :, :], w2_top, preferred_element_type=F32
    ).astype(BF)
    out_ref[_rows((my - 1) % N_DEV), :] = jnp.dot(
        rs_l[2, :, :], w2_bot, preferred_element_type=F32
    ).astype(BF)

    t0_r.wait()
    t0_l.wait()
    t1_r, t1_l = ag_send(
        1, my % N_DEV, my % N_DEV,
        h_buf.at[_rows(my), CR], h_buf.at[_rows(my), CL],
    )
    out_ref[_rows(my), :] = (
        jnp.dot(h_buf[_rows(my), CR], w2_top, preferred_element_type=F32)
        + jnp.dot(h_buf[_rows(my), CL], w2_bot, preferred_element_type=F32)
    ).astype(BF)

    t1_r.wait()
    t1_l.wait()
    t2_r, t2_l = ag_send(
        2, (my - 1) % N_DEV, (my + 1) % N_DEV,
        h_buf.at[_rows((my - 1) % N_DEV), CR],
        h_buf.at[_rows((my + 1) % N_DEV), CL],
    )
    out_ref[_rows((my - 1) % N_DEV), :] = (
        out_ref[_rows((my - 1) % N_DEV), :]
        + jnp.dot(h_buf[_rows((my - 1) % N_DEV), CR], w2_top,
                  preferred_element_type=F32)
    ).astype(BF)
    out_ref[_rows((my + 1) % N_DEV), :] = (
        out_ref[_rows((my + 1) % N_DEV), :]
        + jnp.dot(h_buf[_rows((my + 1) % N_DEV), CL], w2_bot,
                  preferred_element_type=F32)
    ).astype(BF)

    t2_r.wait()
    t2_l.wait()
    c2 = (my + 2) % N_DEV
    out_ref[_rows(c2), :] = (
        jnp.dot(h_buf[_rows(c2), CR], w2_top, preferred_element_type=F32)
        + jnp.dot(h_buf[_rows(c2), CL], w2_bot, preferred_element_type=F32)
    ).astype(BF)


def kernel(x, W1, W2):
    xb = x.astype(BF)
    W1b = W1.astype(BF)
    W2b = W2.astype(BF)

    sem3 = pltpu.SemaphoreType.DMA((N_DEV - 1,))
    return pl.pallas_call(
        _body,
        out_shape=jax.ShapeDtypeStruct((M, D), BF),
        in_specs=[
            pl.BlockSpec(memory_space=pltpu.VMEM),
            pl.BlockSpec(memory_space=pltpu.VMEM),
            pl.BlockSpec(memory_space=pltpu.VMEM),
        ],
        out_specs=pl.BlockSpec(memory_space=pltpu.VMEM),
        scratch_shapes=[
            pltpu.VMEM((M, D), BF),
            pltpu.VMEM((N_DEV - 1, M_BLK, D_HALF), BF),
            pltpu.VMEM((N_DEV - 1, M_BLK, D_HALF), BF),
            sem3, sem3, sem3, sem3,
            sem3, sem3, sem3, sem3,
        ],
        compiler_params=pltpu.CompilerParams(collective_id=0),
    )(xb, W1b, W2b)
